# baseline (speedup 1.0000x reference)
import functools

import jax
import jax.numpy as jnp
from jax import lax
from jax.experimental import pallas as pl
from jax.experimental.pallas import tpu as pltpu
from jax.experimental.pallas import tpu_sc as plsc

N = 16384
C = 1000

_info = plsc.get_sparse_core_info()
_NC, _NS = _info.num_cores, _info.num_subcores
_NW = _NC * _NS
_SPLIT = 12288
_DPW = (N - _SPLIT) // _NW
_PPW = N // _NW
_CR = 40
_NCHUNK = C // _CR
_NPAIR = _NCHUNK // 2
_NG = _PPW // 16
_NACC = 8
_TCBLK = 4096


def _sc_part(yht, y):
    mesh = plsc.VectorSubcoreMesh(core_axis_name="c", subcore_axis_name="s")

    @functools.partial(
        pl.kernel,
        mesh=mesh,
        out_type=jax.ShapeDtypeStruct((_NW, 16), jnp.float32),
        scratch_types=[
            pltpu.VMEM((_CR, _DPW), jnp.float32),
            pltpu.VMEM((_CR, _DPW), jnp.float32),
            pltpu.VMEM((_PPW,), jnp.int32),
            pltpu.VMEM((_NG, 16, 128), jnp.float32),
            pltpu.VMEM((16,), jnp.float32),
            pltpu.SemaphoreType.DMA,
            pltpu.SemaphoreType.DMA,
            pltpu.SemaphoreType.DMA,
        ],
    )
    def k(yht_hbm, y_hbm, out_hbm, buf0, buf1, y_v, picks_v, acc_v,
          sem0, sem1, psem):
        wid = lax.axis_index("s") * _NC + lax.axis_index("c")
        pcol0 = wid * _PPW
        dcol0 = _SPLIT + wid * _DPW
        pltpu.sync_copy(y_hbm.at[pl.ds(pcol0, _PPW)], y_v)

        for g in range(_NG):
            y16 = y_v[pl.ds(g * 16, 16)]
            pltpu.async_copy(
                yht_hbm.at[y16, pl.ds(pcol0 + (g // 8) * 128, 128)],
                picks_v.at[g],
                psem,
            )

        def start(ch, buf, sem):
            pltpu.async_copy(
                yht_hbm.at[pl.ds(ch * _CR, _CR), pl.ds(dcol0, _DPW)], buf, sem
            )

        def drain(buf, sem):
            pltpu.make_async_copy(
                yht_hbm.at[pl.ds(0, _CR), pl.ds(0, _DPW)], buf, sem
            ).wait()

        def consume(buf, carry):
            def row_body(rg, aa):
                aa = list(aa)
                for q in range(2):
                    for s in range(_DPW // 16):
                        x = buf[rg * 2 + q, pl.ds(s * 16, 16)]
                        j = q * (_DPW // 16) + s
                        aa[j % _NACC] = aa[j % _NACC] + x
                return tuple(aa)

            return lax.fori_loop(0, _CR // 2, row_body, carry)

        start(0, buf0, sem0)

        def pair_body(p, carry):
            ch0 = p * 2
            start(ch0 + 1, buf1, sem1)
            drain(buf0, sem0)
            carry = consume(buf0, carry)
            start(ch0 + 2, buf0, sem0)
            drain(buf1, sem1)
            carry = consume(buf1, carry)
            return carry

        zero = jnp.zeros((16,), jnp.float32)
        carry = lax.fori_loop(0, _NPAIR, pair_body, tuple([zero] * _NACC))
        drain(buf0, sem0)
        carry = consume(buf0, carry)

        for g in range(_NG):
            pltpu.make_async_copy(
                yht_hbm.at[pl.ds(0, 16), pl.ds(0, 128)], picks_v.at[g], psem
            ).wait()

        lane = lax.iota(jnp.int32, 16)
        gacc = jnp.zeros((16,), jnp.float32)
        for g in range(_NG):
            off = (g % 8) * 16
            for kk in range(16):
                gacc = gacc + jnp.where(
                    lane == kk, picks_v[g, kk, pl.ds(off, 16)], 0.0
                )

        acc = carry[0]
        for a in carry[1:]:
            acc = acc + a
        acc_v[...] = acc - 2.0 * gacc
        pltpu.sync_copy(acc_v, out_hbm.at[wid])

    return k(yht, y)


def _tc_part(yht):
    def body(x_ref, o_ref):
        @pl.when(pl.program_id(0) == 0)
        def _():
            o_ref[0, 0] = 0.0

        o_ref[0, 0] += jnp.sum(x_ref[...])

    return pl.pallas_call(
        body,
        grid=(_SPLIT // _TCBLK,),
        in_specs=[pl.BlockSpec((C, _TCBLK), lambda i: (0, i))],
        out_specs=pl.BlockSpec(memory_space=pltpu.SMEM),
        out_shape=jax.ShapeDtypeStruct((1, 1), jnp.float32),
    )(yht)


def kernel(yh, y):
    yht = yh.T
    partials = _sc_part(yht, y.astype(jnp.int32))
    dense = _tc_part(yht)
    return dense[0, 0] + partials.sum()


# speedup vs baseline: 1.0729x; 1.0729x over previous
import functools

import jax
import jax.numpy as jnp
from jax import lax
from jax.experimental import pallas as pl
from jax.experimental.pallas import tpu as pltpu
from jax.experimental.pallas import tpu_sc as plsc

N = 16384
C = 1000

_info = plsc.get_sparse_core_info()
_NC, _NS = _info.num_cores, _info.num_subcores
_NW = _NC * _NS
_SPLIT = 12288
_DPW = (N - _SPLIT) // _NW
_PPW = N // _NW
_CR = 200
_NCHUNK = C // _CR
_NPAIR = _NCHUNK // 2
_NG = _PPW // 16
_NACC = 8
_TCBLK = 4096


def _sc_part(yht, y):
    mesh = plsc.VectorSubcoreMesh(core_axis_name="c", subcore_axis_name="s")

    @functools.partial(
        pl.kernel,
        mesh=mesh,
        out_type=jax.ShapeDtypeStruct((_NW, 16), jnp.float32),
        scratch_types=[
            pltpu.VMEM((_CR, _DPW), jnp.float32),
            pltpu.VMEM((_CR, _DPW), jnp.float32),
            pltpu.VMEM((_PPW,), jnp.int32),
            pltpu.VMEM((_NG, 16, 128), jnp.float32),
            pltpu.VMEM((16,), jnp.float32),
            pltpu.SemaphoreType.DMA,
            pltpu.SemaphoreType.DMA,
            pltpu.SemaphoreType.DMA,
        ],
    )
    def k(yht_hbm, y_hbm, out_hbm, buf0, buf1, y_v, picks_v, acc_v,
          sem0, sem1, psem):
        wid = lax.axis_index("s") * _NC + lax.axis_index("c")
        pcol0 = wid * _PPW
        dcol0 = _SPLIT + wid * _DPW
        pltpu.sync_copy(y_hbm.at[pl.ds(pcol0, _PPW)], y_v)

        for g in range(_NG):
            y16 = y_v[pl.ds(g * 16, 16)]
            pltpu.async_copy(
                yht_hbm.at[y16, pl.ds(pcol0 + (g // 8) * 128, 128)],
                picks_v.at[g],
                psem,
            )

        def start(ch, buf, sem):
            pltpu.async_copy(
                yht_hbm.at[pl.ds(ch * _CR, _CR), pl.ds(dcol0, _DPW)], buf, sem
            )

        def drain(buf, sem):
            pltpu.make_async_copy(
                yht_hbm.at[pl.ds(0, _CR), pl.ds(0, _DPW)], buf, sem
            ).wait()

        def consume(buf, carry):
            def row_body(rg, aa):
                aa = list(aa)
                for q in range(2):
                    for s in range(_DPW // 16):
                        x = buf[rg * 2 + q, pl.ds(s * 16, 16)]
                        j = q * (_DPW // 16) + s
                        aa[j % _NACC] = aa[j % _NACC] + x
                return tuple(aa)

            return lax.fori_loop(0, _CR // 2, row_body, carry)

        start(0, buf0, sem0)

        def pair_body(p, carry):
            ch0 = p * 2
            start(ch0 + 1, buf1, sem1)
            drain(buf0, sem0)
            carry = consume(buf0, carry)
            start(ch0 + 2, buf0, sem0)
            drain(buf1, sem1)
            carry = consume(buf1, carry)
            return carry

        zero = jnp.zeros((16,), jnp.float32)
        carry = lax.fori_loop(0, _NPAIR, pair_body, tuple([zero] * _NACC))
        drain(buf0, sem0)
        carry = consume(buf0, carry)

        for g in range(_NG):
            pltpu.make_async_copy(
                yht_hbm.at[pl.ds(0, 16), pl.ds(0, 128)], picks_v.at[g], psem
            ).wait()

        lane = lax.iota(jnp.int32, 16)
        gacc = jnp.zeros((16,), jnp.float32)
        for g in range(_NG):
            off = (g % 8) * 16
            for kk in range(16):
                gacc = gacc + jnp.where(
                    lane == kk, picks_v[g, kk, pl.ds(off, 16)], 0.0
                )

        acc = carry[0]
        for a in carry[1:]:
            acc = acc + a
        acc_v[...] = acc - 2.0 * gacc
        pltpu.sync_copy(acc_v, out_hbm.at[wid])

    return k(yht, y)


def _tc_part(yht):
    def body(x_ref, o_ref):
        @pl.when(pl.program_id(0) == 0)
        def _():
            o_ref[0, 0] = 0.0

        o_ref[0, 0] += jnp.sum(x_ref[...])

    return pl.pallas_call(
        body,
        grid=(_SPLIT // _TCBLK,),
        in_specs=[pl.BlockSpec((C, _TCBLK), lambda i: (0, i))],
        out_specs=pl.BlockSpec(memory_space=pltpu.SMEM),
        out_shape=jax.ShapeDtypeStruct((1, 1), jnp.float32),
    )(yht)


def kernel(yh, y):
    yht = yh.T
    partials = _sc_part(yht, y.astype(jnp.int32))
    dense = _tc_part(yht)
    return dense[0, 0] + partials.sum()


# speedup vs baseline: 1.0911x; 1.0169x over previous
import functools

import jax
import jax.numpy as jnp
from jax import lax
from jax.experimental import pallas as pl
from jax.experimental.pallas import tpu as pltpu
from jax.experimental.pallas import tpu_sc as plsc

N = 16384
C = 1000

_info = plsc.get_sparse_core_info()
_NC, _NS = _info.num_cores, _info.num_subcores
_NW = _NC * _NS
_SPLIT = 12288
_DPW = (N - _SPLIT) // _NW
_PPW = N // _NW
_CR = 200
_NCHUNK = C // _CR
_NPAIR = _NCHUNK // 2
_NG = _PPW // 16
_NACC = 8
_TCBLK = 4096


def _sc_part(yht, y):
    mesh = plsc.VectorSubcoreMesh(core_axis_name="c", subcore_axis_name="s")

    @functools.partial(
        pl.kernel,
        mesh=mesh,
        out_type=jax.ShapeDtypeStruct((_NW, 16), jnp.float32),
        scratch_types=[
            pltpu.VMEM((_CR, _DPW), jnp.float32),
            pltpu.VMEM((_CR, _DPW), jnp.float32),
            pltpu.VMEM((_PPW,), jnp.int32),
            pltpu.VMEM((_NG, 16, 128), jnp.float32),
            pltpu.VMEM((16,), jnp.float32),
            pltpu.SemaphoreType.DMA,
            pltpu.SemaphoreType.DMA,
            pltpu.SemaphoreType.DMA,
        ],
    )
    def k(yht_hbm, y_hbm, out_hbm, buf0, buf1, y_v, picks_v, acc_v,
          sem0, sem1, psem):
        wid = lax.axis_index("s") * _NC + lax.axis_index("c")
        pcol0 = wid * _PPW
        dcol0 = _SPLIT + wid * _DPW
        pltpu.sync_copy(y_hbm.at[pl.ds(pcol0, _PPW)], y_v)

        for g in range(_NG):
            y16 = y_v[pl.ds(g * 16, 16)]
            pltpu.async_copy(
                yht_hbm.at[y16, pl.ds(pcol0 + (g // 8) * 128, 128)],
                picks_v.at[g],
                psem,
            )

        def start(ch, buf, sem):
            pltpu.async_copy(
                yht_hbm.at[pl.ds(ch * _CR, _CR), pl.ds(dcol0, _DPW)], buf, sem
            )

        def drain(buf, sem):
            pltpu.make_async_copy(
                yht_hbm.at[pl.ds(0, _CR), pl.ds(0, _DPW)], buf, sem
            ).wait()

        def consume(buf, carry):
            def row_body(rg, aa):
                aa = list(aa)
                for q in range(2):
                    for s in range(_DPW // 16):
                        x = buf[rg * 2 + q, pl.ds(s * 16, 16)]
                        j = q * (_DPW // 16) + s
                        aa[j % _NACC] = aa[j % _NACC] + x
                return tuple(aa)

            return lax.fori_loop(0, _CR // 2, row_body, carry)

        start(0, buf0, sem0)

        def pair_body(p, carry):
            ch0 = p * 2
            start(ch0 + 1, buf1, sem1)
            drain(buf0, sem0)
            carry = consume(buf0, carry)
            start(ch0 + 2, buf0, sem0)
            drain(buf1, sem1)
            carry = consume(buf1, carry)
            return carry

        zero = jnp.zeros((16,), jnp.float32)
        carry = lax.fori_loop(0, _NPAIR, pair_body, tuple([zero] * _NACC))
        drain(buf0, sem0)
        carry = consume(buf0, carry)

        for g in range(_NG):
            pltpu.make_async_copy(
                yht_hbm.at[pl.ds(0, 16), pl.ds(0, 128)], picks_v.at[g], psem
            ).wait()

        lane = lax.iota(jnp.int32, 16)
        gacc = jnp.zeros((16,), jnp.float32)
        for g in range(_NG):
            off = (g % 8) * 16
            for kk in range(16):
                gacc = gacc + jnp.where(
                    lane == kk, picks_v[g, kk, pl.ds(off, 16)], 0.0
                )

        acc = carry[0]
        for a in carry[1:]:
            acc = acc + a
        acc_v[...] = acc - 2.0 * gacc
        pltpu.sync_copy(acc_v, out_hbm.at[wid])

    return k(yht, y)


def _tc_part(yht):
    def body(x_ref, o_ref):
        @pl.when(pl.program_id(0) == 0)
        def _():
            o_ref[...] = jnp.zeros_like(o_ref)

        o_ref[...] += jnp.sum(x_ref[...], axis=0, keepdims=True)

    return pl.pallas_call(
        body,
        grid=(_SPLIT // _TCBLK,),
        in_specs=[pl.BlockSpec((C, _TCBLK), lambda i: (0, i))],
        out_specs=pl.BlockSpec((1, _TCBLK), lambda i: (0, 0)),
        out_shape=jax.ShapeDtypeStruct((1, _TCBLK), jnp.float32),
    )(yht)


def kernel(yh, y):
    yht = yh.T
    partials = _sc_part(yht, y.astype(jnp.int32))
    dense = _tc_part(yht)
    return dense.sum() + partials.sum()
